# Initial kernel scaffold; baseline (speedup 1.0000x reference)
import jax
import jax.numpy as jnp
from jax.experimental import pallas as pl


def kernel(boxes, scores):
    raise NotImplementedError("write your pallas kernel here")


# speedup vs baseline: 20.8886x; 20.8886x over previous
import jax
import jax.numpy as jnp
from jax.experimental import pallas as pl
from jax.experimental.pallas import tpu as pltpu

_IOU_THR = 0.5
_SCORE_THR = 0.05
_MAX_OUT = 300
_N = 5000
_B = 512
_NP = 5120
_NBLK = _NP // _B


def _iou_block(bc_ref, br_ref, rbase, cbase, cw):
    x1p = bc_ref[rbase:rbase + _B, 0:1]
    y1p = bc_ref[rbase:rbase + _B, 1:2]
    x2p = bc_ref[rbase:rbase + _B, 2:3]
    y2p = bc_ref[rbase:rbase + _B, 3:4]
    area_p = (x2p - x1p) * (y2p - y1p)
    x1t = br_ref[0:1, cbase:cbase + cw]
    y1t = br_ref[1:2, cbase:cbase + cw]
    x2t = br_ref[2:3, cbase:cbase + cw]
    y2t = br_ref[3:4, cbase:cbase + cw]
    area_t = (x2t - x1t) * (y2t - y1t)
    w = jnp.clip(jnp.minimum(x2p, x2t) - jnp.maximum(x1p, x1t), 0.0)
    h = jnp.clip(jnp.minimum(y2p, y2t) - jnp.maximum(y1p, y1t), 0.0)
    inter = w * h
    union = area_p + area_t - inter
    return inter / jnp.maximum(union, 1e-8)


def _nms_kernel(bc_ref, br_ref, s_ref, keep_ref, iou_ref):
    keep_ref[...] = (s_ref[...] > _SCORE_THR).astype(jnp.float32)
    lane = jax.lax.broadcasted_iota(jnp.int32, (1, _B), 1)

    for p in range(_NBLK):
        base = p * _B
        iou_ref[...] = _iou_block(bc_ref, br_ref, base, base, _B)

        def scan_body(r, _, base=base):
            row = iou_ref[pl.ds(r, 1), :]
            kblk = keep_ref[0:1, base:base + _B]
            kr = jnp.max(jnp.where(lane == r, kblk, 0.0),
                         axis=1, keepdims=True)
            sup = ((row > _IOU_THR) & (lane > r)).astype(jnp.float32) * kr
            keep_ref[0:1, base:base + _B] = kblk * (1.0 - sup)
            return 0

        jax.lax.fori_loop(0, _B, scan_body, 0)

        tail = _NP - base - _B
        if tail > 0:
            kblk = keep_ref[0:1, base:base + _B]
            r_io = jax.lax.broadcasted_iota(jnp.int32, (_B, _B), 0)
            c_io = jax.lax.broadcasted_iota(jnp.int32, (_B, _B), 1)
            eye = (r_io == c_io).astype(jnp.float32)
            kcol = jax.lax.dot_general(
                eye, kblk, (((1,), (1,)), ((), ())),
                preferred_element_type=jnp.float32)
            iou_t = _iou_block(bc_ref, br_ref, base, base + _B, tail)
            sup = jnp.max(iou_t * kcol, axis=0, keepdims=True)
            kt = keep_ref[0:1, base + _B:]
            keep_ref[0:1, base + _B:] = kt * (sup <= _IOU_THR).astype(
                jnp.float32)


def _run_nms(bc, br, s_row):
    return pl.pallas_call(
        _nms_kernel,
        out_shape=jax.ShapeDtypeStruct((1, _NP), jnp.float32),
        scratch_shapes=[pltpu.VMEM((_B, _B), jnp.float32)],
    )(bc, br, s_row)


def kernel(boxes, scores):
    order = jnp.argsort(-scores)
    b = boxes[order]
    s = scores[order]
    pad = _NP - _N
    bp = jnp.pad(b, ((0, pad), (0, 0)))
    sp = jnp.pad(s, ((0, pad),), constant_values=-1.0)
    keep = _run_nms(bp, bp.T, sp[None, :])[0]

    masked = jnp.where(keep > 0.5, sp, -1.0)
    topk_s, topk_i = jax.lax.top_k(masked, _MAX_OUT)
    valid = topk_s > _SCORE_THR
    final_scores = jnp.where(valid, topk_s, 0.0)
    final_boxes = bp[topk_i] * valid[:, None].astype(bp.dtype)
    final_class_idx = jnp.where(valid, 0, -1)
    return final_scores, final_class_idx, final_boxes


# speedup vs baseline: 102.8607x; 4.9243x over previous
import jax
import jax.numpy as jnp
from jax.experimental import pallas as pl
from jax.experimental.pallas import tpu as pltpu

_IOU_THR = 0.5
_SCORE_THR = 0.05
_MAX_OUT = 300
_N = 5000
_B = 512
_NP = 5120
_NBLK = _NP // _B

_DN = (((1,), (0,)), ((), ()))


def _iou_block(bc_ref, br_ref, rbase, cbase, cw):
    x1p = bc_ref[rbase:rbase + _B, 0:1]
    y1p = bc_ref[rbase:rbase + _B, 1:2]
    x2p = bc_ref[rbase:rbase + _B, 2:3]
    y2p = bc_ref[rbase:rbase + _B, 3:4]
    area_p = (x2p - x1p) * (y2p - y1p)
    x1t = br_ref[0:1, cbase:cbase + cw]
    y1t = br_ref[1:2, cbase:cbase + cw]
    x2t = br_ref[2:3, cbase:cbase + cw]
    y2t = br_ref[3:4, cbase:cbase + cw]
    area_t = (x2t - x1t) * (y2t - y1t)
    w = jnp.clip(jnp.minimum(x2p, x2t) - jnp.maximum(x1p, x1t), 0.0)
    h = jnp.clip(jnp.minimum(y2p, y2t) - jnp.maximum(y1p, y1t), 0.0)
    inter = w * h
    union = area_p + area_t - inter
    return inter / jnp.maximum(union, 1e-8)


def _nms_kernel(bc_ref, br_ref, sr_ref, sc_ref, sco_ref, box_ref, a_ref):
    ri = jax.lax.broadcasted_iota(jnp.int32, (_B, _B), 0)
    ci = jax.lax.broadcasted_iota(jnp.int32, (_B, _B), 1)
    upper = ri < ci

    kb = [(sr_ref[0:1, q * _B:(q + 1) * _B] > _SCORE_THR).astype(jnp.float32)
          for q in range(_NBLK)]

    for p in range(_NBLK):
        base = p * _B
        iou_pp = _iou_block(bc_ref, br_ref, base, base, _B)
        a_ref[...] = ((iou_pp > _IOU_THR) & upper).astype(jnp.float32)

        keep0 = kb[p]

        def cond(carry):
            return carry[1]

        def body(carry):
            k, _ = carry
            cnt = jax.lax.dot_general(
                k, a_ref[...], _DN, preferred_element_type=jnp.float32)
            kn = jnp.where(cnt == 0.0, keep0, 0.0)
            changed = jnp.any(kn != k)
            return kn, changed

        kblk, _ = jax.lax.while_loop(cond, body, (keep0, True))
        kb[p] = kblk

        tail = _NP - base - _B
        if tail > 0:
            iou_pt = _iou_block(bc_ref, br_ref, base, base + _B, tail)
            a_pt = (iou_pt > _IOU_THR).astype(jnp.float32)
            cnt = jax.lax.dot_general(
                kblk, a_pt, _DN, preferred_element_type=jnp.float32)
            for q in range(p + 1, _NBLK):
                sl = cnt[0:1, (q - p - 1) * _B:(q - p) * _B]
                kb[q] = jnp.where(sl == 0.0, kb[q], 0.0)

    tri = (ri <= ci).astype(jnp.float32)
    ranks = []
    offset = jnp.zeros((1, 1), jnp.float32)
    for p in range(_NBLK):
        rblk = jax.lax.dot_general(
            kb[p], tri, _DN, preferred_element_type=jnp.float32) + offset
        offset = offset + jnp.sum(kb[p], keepdims=True)
        ranks.append(rblk)
    rank = jnp.concatenate(ranks, axis=1)
    keep = jnp.concatenate(kb, axis=1)

    kf = (jax.lax.broadcasted_iota(jnp.int32, (_B, 1), 0) + 1).astype(
        jnp.float32)
    sel = ((rank == kf) & (keep > 0.5)).astype(jnp.float32)
    sco_ref[...] = jax.lax.dot_general(
        sel, sc_ref[...], _DN, preferred_element_type=jnp.float32,
        precision=jax.lax.Precision.HIGHEST)
    box_ref[...] = jax.lax.dot_general(
        sel, bc_ref[...], _DN, preferred_element_type=jnp.float32,
        precision=jax.lax.Precision.HIGHEST)


def _run_nms(bc, br, s_row, s_col):
    return pl.pallas_call(
        _nms_kernel,
        out_shape=(jax.ShapeDtypeStruct((_B, 1), jnp.float32),
                   jax.ShapeDtypeStruct((_B, 4), jnp.float32)),
        scratch_shapes=[pltpu.VMEM((_B, _B), jnp.float32)],
    )(bc, br, s_row, s_col)


def kernel(boxes, scores):
    order = jnp.argsort(-scores)
    b = boxes[order]
    s = scores[order]
    pad = _NP - _N
    bp = jnp.pad(b, ((0, pad), (0, 0)))
    sp = jnp.pad(s, ((0, pad),), constant_values=-1.0)
    sco, box = _run_nms(bp, bp.T, sp[None, :], sp[:, None])

    final_scores = sco[:_MAX_OUT, 0]
    final_boxes = box[:_MAX_OUT]
    valid = final_scores > _SCORE_THR
    final_class_idx = jnp.where(valid, 0, -1)
    return final_scores, final_class_idx, final_boxes


# speedup vs baseline: 111.9876x; 1.0887x over previous
import jax
import jax.numpy as jnp
from jax.experimental import pallas as pl
from jax.experimental.pallas import tpu as pltpu

_IOU_THR = 0.5
_SCORE_THR = 0.05
_MAX_OUT = 300
_N = 5000
_B = 512
_NP = 5120
_NBLK = _NP // _B

_DN = (((1,), (0,)), ((), ()))


def _iou_block(bc_ref, br_ref, rbase, cbase, cw):
    x1p = bc_ref[rbase:rbase + _B, 0:1]
    y1p = bc_ref[rbase:rbase + _B, 1:2]
    x2p = bc_ref[rbase:rbase + _B, 2:3]
    y2p = bc_ref[rbase:rbase + _B, 3:4]
    area_p = (x2p - x1p) * (y2p - y1p)
    x1t = br_ref[0:1, cbase:cbase + cw]
    y1t = br_ref[1:2, cbase:cbase + cw]
    x2t = br_ref[2:3, cbase:cbase + cw]
    y2t = br_ref[3:4, cbase:cbase + cw]
    area_t = (x2t - x1t) * (y2t - y1t)
    w = jnp.clip(jnp.minimum(x2p, x2t) - jnp.maximum(x1p, x1t), 0.0)
    h = jnp.clip(jnp.minimum(y2p, y2t) - jnp.maximum(y1p, y1t), 0.0)
    inter = w * h
    union = area_p + area_t - inter
    return inter / jnp.maximum(union, 1e-8)


def _nms_kernel(bc_ref, br_ref, sr_ref, sc_ref, sco_ref, box_ref, a_ref):
    ri = jax.lax.broadcasted_iota(jnp.int32, (_B, _B), 0)
    ci = jax.lax.broadcasted_iota(jnp.int32, (_B, _B), 1)
    upper = ri < ci

    kb = [(sr_ref[0:1, q * _B:(q + 1) * _B] > _SCORE_THR).astype(jnp.float32)
          for q in range(_NBLK)]

    for p in range(_NBLK):
        base = p * _B
        iou_pp = _iou_block(bc_ref, br_ref, base, base, _B)
        a_ref[...] = ((iou_pp > _IOU_THR) & upper).astype(jnp.float32)

        keep0 = kb[p]

        def cond(carry):
            return carry[1]

        def body(carry):
            k, _ = carry
            cnt = jax.lax.dot_general(
                k, a_ref[...], _DN, preferred_element_type=jnp.float32)
            kn = jnp.where(cnt == 0.0, keep0, 0.0)
            changed = jnp.any(kn != k)
            return kn, changed

        kblk, _ = jax.lax.while_loop(cond, body, (keep0, True))
        kb[p] = kblk

        tail = _NP - base - _B
        if tail > 0:
            iou_pt = _iou_block(bc_ref, br_ref, base, base + _B, tail)
            a_pt = (iou_pt > _IOU_THR).astype(jnp.float32)
            cnt = jax.lax.dot_general(
                kblk, a_pt, _DN, preferred_element_type=jnp.float32)
            for q in range(p + 1, _NBLK):
                sl = cnt[0:1, (q - p - 1) * _B:(q - p) * _B]
                kb[q] = jnp.where(sl == 0.0, kb[q], 0.0)

    tri = (ri <= ci).astype(jnp.float32)
    ranks = []
    offset = jnp.zeros((1, 1), jnp.float32)
    for p in range(_NBLK):
        rblk = jax.lax.dot_general(
            kb[p], tri, _DN, preferred_element_type=jnp.float32) + offset
        offset = offset + jnp.sum(kb[p], keepdims=True)
        ranks.append(rblk)
    rank = jnp.concatenate(ranks, axis=1)
    keep = jnp.concatenate(kb, axis=1)

    kf = (jax.lax.broadcasted_iota(jnp.int32, (_B, 1), 0) + 1).astype(
        jnp.float32)
    sel = ((rank == kf) & (keep > 0.5)).astype(jnp.float32)
    sco_ref[...] = jax.lax.dot_general(
        sel, sc_ref[...], _DN, preferred_element_type=jnp.float32,
        precision=jax.lax.Precision.HIGHEST)
    box_ref[...] = jax.lax.dot_general(
        sel, bc_ref[...], _DN, preferred_element_type=jnp.float32,
        precision=jax.lax.Precision.HIGHEST)


def _run_nms(bc, br, s_row, s_col):
    return pl.pallas_call(
        _nms_kernel,
        out_shape=(jax.ShapeDtypeStruct((_B, 1), jnp.float32),
                   jax.ShapeDtypeStruct((_B, 4), jnp.float32)),
        scratch_shapes=[pltpu.VMEM((_B, _B), jnp.float32)],
    )(bc, br, s_row, s_col)


def kernel(boxes, scores):
    order = jnp.argsort(-scores)
    g = jnp.concatenate([boxes, scores[:, None]], axis=1)[order]
    pad = _NP - _N
    bp = jnp.pad(g[:, :4], ((0, pad), (0, 0)))
    sp = jnp.pad(g[:, 4], ((0, pad),), constant_values=-1.0)
    sco, box = _run_nms(bp, bp.T, sp[None, :], sp[:, None])

    final_scores = sco[:_MAX_OUT, 0]
    final_boxes = box[:_MAX_OUT]
    valid = final_scores > _SCORE_THR
    final_class_idx = jnp.where(valid, 0, -1)
    return final_scores, final_class_idx, final_boxes
